# baseline (speedup 1.0000x reference)
import functools

import jax
import jax.numpy as jnp
from jax import lax
from jax.experimental import pallas as pl
from jax.experimental.pallas import tpu as pltpu
from jax.experimental.pallas import tpu_sc as plsc

B = 64
H = 2048
V = 32000
K = 8
CH = 1600
NB = 4
NC = V // CH
NCAND = 256
NEG_INF = float("-inf")


def _pack_cols(cols, n, dtype):
    io = lax.broadcasted_iota(jnp.int32, (B, n), 1)
    acc = jnp.zeros((B, n), dtype)
    for j, c in enumerate(cols):
        acc = jnp.where(io == j, c.astype(dtype), acc)
    return acc


def _topk_body(h_ref, w_hbm, vals_out, idx_out, wbuf, cv_ref, ci_ref, sems):
    for b in range(NB):
        pltpu.async_copy(w_hbm.at[pl.ds(b * CH, CH), :], wbuf.at[b],
                         sems.at[b])

    cv_ref[...] = jnp.full((B, NCAND), NEG_INF, jnp.float32)
    ci_ref[...] = jnp.zeros((B, NCAND), jnp.int32)

    h = h_ref[...]
    m = jnp.full((B, 1), NEG_INF, jnp.float32)
    s = jnp.zeros((B, 1), jnp.float32)
    io = lax.broadcasted_iota(jnp.int32, (B, CH), 1)

    for c in range(NC):
        slot = c % NB
        pltpu.make_async_copy(w_hbm.at[pl.ds(c * CH, CH), :], wbuf.at[slot],
                              sems.at[slot]).wait()
        w = wbuf[slot]
        logits = lax.dot_general(
            h, w, (((1,), (1,)), ((), ())),
            preferred_element_type=jnp.float32)
        nxt = c + NB
        if nxt < NC:
            pltpu.async_copy(w_hbm.at[pl.ds(nxt * CH, CH), :], wbuf.at[slot],
                             sems.at[slot])

        tmax = jnp.max(logits, axis=1, keepdims=True)
        m_new = jnp.maximum(m, tmax)
        s = s * jnp.exp(m - m_new) + jnp.sum(
            jnp.exp(logits - m_new), axis=1, keepdims=True)
        m = m_new

        x = logits
        for j in range(K):
            mx = jnp.max(x, axis=1, keepdims=True)
            loc = jnp.min(jnp.where(x == mx, io, CH), axis=1, keepdims=True)
            col = c * K + j
            cv_ref[:, col:col + 1] = mx
            ci_ref[:, col:col + 1] = loc + c * CH
            x = jnp.where(io == loc, NEG_INF, x)

    lse = m + jnp.log(s)
    comb_v = cv_ref[...]
    comb_i = ci_ref[...]
    io2 = lax.broadcasted_iota(jnp.int32, (B, NCAND), 1)
    new_v, new_i = [], []
    for j in range(K):
        mx = jnp.max(comb_v, axis=1, keepdims=True)
        loc = jnp.min(jnp.where(comb_v == mx, io2, NCAND), axis=1,
                      keepdims=True)
        sel = io2 == loc
        gidx = jnp.max(jnp.where(sel, comb_i, -1), axis=1, keepdims=True)
        new_v.append(mx)
        new_i.append(gidx)
        comb_v = jnp.where(sel, NEG_INF, comb_v)
    vals_out[...] = _pack_cols(new_v, K, jnp.float32) - lse
    idx_out[...] = _pack_cols(new_i, K, jnp.int32)


def _topk_tc(hidden, w_lm):
    return pl.pallas_call(
        _topk_body,
        in_specs=[
            pl.BlockSpec((B, H), lambda: (0, 0)),
            pl.BlockSpec(memory_space=pltpu.MemorySpace.HBM),
        ],
        out_specs=[
            pl.BlockSpec((B, K), lambda: (0, 0)),
            pl.BlockSpec((B, K), lambda: (0, 0)),
        ],
        out_shape=[
            jax.ShapeDtypeStruct((B, K), jnp.float32),
            jax.ShapeDtypeStruct((B, K), jnp.int32),
        ],
        scratch_shapes=[
            pltpu.VMEM((NB, CH, H), jnp.float32),
            pltpu.VMEM((B, NCAND), jnp.float32),
            pltpu.VMEM((B, NCAND), jnp.int32),
            pltpu.SemaphoreType.DMA((NB,)),
        ],
    )(hidden, w_lm)


def _d2t_map_sc(d2t, idx_flat):
    n = idx_flat.shape[0]
    mesh = plsc.VectorSubcoreMesh(core_axis_name="c", subcore_axis_name="s")

    @functools.partial(
        pl.kernel,
        mesh=mesh,
        out_type=jax.ShapeDtypeStruct((n,), jnp.int32),
        scratch_types=[
            pltpu.VMEM((16,), jnp.int32),
            pltpu.VMEM((16,), jnp.int32),
            pltpu.SemaphoreType.DMA,
        ],
    )
    def k(d2t_hbm, idx_hbm, out_hbm, idx_v, g_v, sem):
        wid = lax.axis_index("s") * 2 + lax.axis_index("c")
        base = wid * 16
        pltpu.sync_copy(idx_hbm.at[pl.ds(base, 16)], idx_v)
        pltpu.async_copy(d2t_hbm.at[idx_v], g_v, sem).wait()
        g_v[...] = g_v[...] + idx_v[...]
        pltpu.sync_copy(g_v, out_hbm.at[pl.ds(base, 16)])

    return k(d2t, idx_flat)


def kernel(hidden_states, d2t, W_lm):
    scores, topk_index = _topk_tc(hidden_states, W_lm)
    mapped = _d2t_map_sc(d2t, topk_index.reshape(B * K)).reshape(B, K)
    return mapped, scores
